# Initial kernel scaffold; baseline (speedup 1.0000x reference)
import jax
import jax.numpy as jnp
from jax.experimental import pallas as pl


def kernel(x, edge_index, Wl0, Wr0, b0, g0, be0, m0, v0, Wl1, Wr1, b1, g1, be1, m1, v1, Wl2, Wr2, b2):
    raise NotImplementedError("write your pallas kernel here")


# speedup vs baseline: 7.3158x; 7.3158x over previous
import jax
import jax.numpy as jnp
from jax import lax
from jax.experimental import pallas as pl
from jax.experimental.pallas import tpu as pltpu
from jax.experimental.pallas import tpu_sc as plsc

N = 10000
E = 320000
D_IN = 128
D_H = 64
D_OUT = 128

NUM_CORES = 2
NUM_SUBCORES = 16
NUM_WORKERS = NUM_CORES * NUM_SUBCORES
CHUNK = 128
NCH = 79
E_PAD = NUM_WORKERS * NCH * CHUNK
EPT = NCH * CHUNK
N_PAD = 10112
RPT = N_PAD // NUM_SUBCORES
CNT_W = 16


def _make_sc_segsum(with_counts):
    mesh = plsc.VectorSubcoreMesh(core_axis_name="c", subcore_axis_name="s",
                                  num_cores=NUM_CORES,
                                  num_subcores=NUM_SUBCORES)
    out_type = [jax.ShapeDtypeStruct((NUM_CORES, N_PAD, D_H), jnp.float32)]
    scratch = [
        pltpu.VMEM((NCH, CHUNK), jnp.int32),
        pltpu.VMEM((NCH, CHUNK), jnp.int32),
        pltpu.VMEM((CHUNK, D_H), jnp.float32),
        pltpu.VMEM_SHARED((N_PAD, D_H), jnp.float32),
        pltpu.SemaphoreType.DMA,
    ]
    if with_counts:
        out_type.append(
            jax.ShapeDtypeStruct((NUM_CORES, N_PAD, CNT_W), jnp.float32))
        scratch += [
            pltpu.VMEM((CHUNK, CNT_W), jnp.float32),
            pltpu.VMEM_SHARED((N_PAD, CNT_W), jnp.float32),
        ]

    def body(p_hbm, srcs_hbm, dsts_hbm, zf_hbm, *rest):
        if with_counts:
            (zc_hbm, ones_hbm, s_out, c_out,
             src_v, dst_v, rows_v, acc, sem, ones_v, cacc) = rest
        else:
            s_out, src_v, dst_v, rows_v, acc, sem = rest
        c = lax.axis_index("c")
        s = lax.axis_index("s")
        wid = c * NUM_SUBCORES + s
        row0 = s * RPT
        pltpu.sync_copy(zf_hbm.at[pl.ds(row0, RPT)], acc.at[pl.ds(row0, RPT)])
        pltpu.sync_copy(srcs_hbm.at[wid], src_v)
        pltpu.sync_copy(dsts_hbm.at[wid], dst_v)
        if with_counts:
            pltpu.sync_copy(zc_hbm.at[pl.ds(row0, RPT)],
                            cacc.at[pl.ds(row0, RPT)])
            pltpu.sync_copy(ones_hbm, ones_v)
        plsc.subcore_barrier()

        def chunk_body(j, carry):
            pltpu.async_copy(p_hbm.at[src_v.at[j]], rows_v, sem).wait()
            pltpu.sync_copy(rows_v, acc.at[dst_v.at[j]], add=True)
            if with_counts:
                pltpu.sync_copy(ones_v, cacc.at[dst_v.at[j]], add=True)
            return carry

        lax.fori_loop(0, NCH, chunk_body, 0)
        plsc.subcore_barrier()
        pltpu.sync_copy(acc.at[pl.ds(row0, RPT)],
                        s_out.at[c, pl.ds(row0, RPT)])
        if with_counts:
            pltpu.sync_copy(cacc.at[pl.ds(row0, RPT)],
                            c_out.at[c, pl.ds(row0, RPT)])

    return pl.kernel(body, out_type=tuple(out_type), mesh=mesh,
                     scratch_types=tuple(scratch),
                     compiler_params=pltpu.CompilerParams(
                         use_tc_tiling_on_sc=False))


def _tc_proj(h, Wl, Wr, b2d):
    n = h.shape[0]
    dout = Wl.shape[1]

    def body(h_ref, wl_ref, wr_ref, b_ref, p_ref, q_ref):
        hv = h_ref[...]
        p_ref[...] = jnp.dot(hv, wl_ref[...],
                             preferred_element_type=jnp.float32)
        q_ref[...] = jnp.dot(hv, wr_ref[...],
                             preferred_element_type=jnp.float32) + b_ref[...]

    return pl.pallas_call(
        body,
        out_shape=(jax.ShapeDtypeStruct((n, dout), jnp.float32),
                   jax.ShapeDtypeStruct((n, dout), jnp.float32)),
    )(h, Wl, Wr, b2d)


def _tc_combine(s_parts, cnt_parts, q, g2, be2, m2, v2):
    def body(s_ref, c_ref, q_ref, g_ref, be_ref, m_ref, v_ref, o_ref):
        ssum = s_ref[0, :N, :] + s_ref[1, :N, :]
        cnt = c_ref[0, :N, 0:1] + c_ref[1, :N, 0:1]
        recip = 1.0 / jnp.maximum(cnt, 1.0)
        t = ssum * recip + q_ref[...]
        scale = g_ref[...] * lax.rsqrt(v_ref[...] + 1e-5)
        o_ref[...] = jnp.maximum((t - m_ref[...]) * scale + be_ref[...], 0.0)

    return pl.pallas_call(
        body, out_shape=jax.ShapeDtypeStruct((N, D_H), jnp.float32),
    )(s_parts, cnt_parts, q, g2, be2, m2, v2)


def _tc_final(s_parts, cnt_parts, h, Wl, Wr, b2d):
    def body(s_ref, c_ref, h_ref, wl_ref, wr_ref, b_ref, o_ref):
        ssum = s_ref[0, :N, :] + s_ref[1, :N, :]
        cnt = c_ref[0, :N, 0:1] + c_ref[1, :N, 0:1]
        agg = ssum * (1.0 / jnp.maximum(cnt, 1.0))
        o_ref[...] = (
            jnp.dot(agg, wl_ref[...], preferred_element_type=jnp.float32)
            + jnp.dot(h_ref[...], wr_ref[...],
                      preferred_element_type=jnp.float32)
            + b_ref[...])

    return pl.pallas_call(
        body, out_shape=jax.ShapeDtypeStruct((N, D_OUT), jnp.float32),
    )(s_parts, cnt_parts, h, Wl, Wr, b2d)


def kernel(x, edge_index, Wl0, Wr0, b0, g0, be0, m0, v0,
           Wl1, Wr1, b1, g1, be1, m1, v1, Wl2, Wr2, b2):
    src = edge_index[0]
    dst = edge_index[1]
    pad = E_PAD - E
    srcs = jnp.concatenate(
        [src, jnp.zeros((pad,), jnp.int32)]).reshape(NUM_WORKERS, NCH, CHUNK)
    dsts = jnp.concatenate(
        [dst, jnp.full((pad,), N, jnp.int32)]).reshape(NUM_WORKERS, NCH, CHUNK)
    zf = jnp.zeros((N_PAD, D_H), jnp.float32)
    zc = jnp.zeros((N_PAD, CNT_W), jnp.float32)
    ones = jnp.ones((CHUNK, CNT_W), jnp.float32)

    seg_cnt = _make_sc_segsum(True)
    seg = _make_sc_segsum(False)

    r2 = lambda a: a.reshape(1, -1)

    p0, q0 = _tc_proj(x, Wl0, Wr0, r2(b0))
    s0, cnt = seg_cnt(p0, srcs, dsts, zf, zc, ones)
    h1 = _tc_combine(s0, cnt, q0, r2(g0), r2(be0), r2(m0), r2(v0))

    p1, q1 = _tc_proj(h1, Wl1, Wr1, r2(b1))
    outs = seg(p1, srcs, dsts, zf)
    s1 = outs[0] if isinstance(outs, (tuple, list)) else outs
    h2 = _tc_combine(s1, cnt, q1, r2(g1), r2(be1), r2(m1), r2(v1))

    outs = seg(h2, srcs, dsts, zf)
    s2 = outs[0] if isinstance(outs, (tuple, list)) else outs
    return _tc_final(s2, cnt, h2, Wl2, Wr2, r2(b2))
